# baseline (speedup 1.0000x reference)
import functools

import jax
import jax.numpy as jnp
from jax import lax
from jax.experimental import pallas as pl
from jax.experimental.pallas import tpu as pltpu
from jax.experimental.pallas import tpu_sc as plsc

VOCAB = 1000000
EMBED_DIM = 64
BATCH = 4096
SEQ_LEN = 200

NC = 2
NS = 16
NW = NC * NS
B_TOTAL = BATCH * SEQ_LEN
N_PER_W = B_TOTAL // NW
IDX_MINOR = 128
R_PER_CHUNK = 8
CHUNK = R_PER_CHUNK * IDX_MINOR
ROWS_PER_W = N_PER_W // IDX_MINOR
T_CHUNKS = ROWS_PER_W // R_PER_CHUNK


def _gather_body(x_hbm, table_hbm, out_hbm, idx_v, rows_v, gsem):
    wid = lax.axis_index("s") * NC + lax.axis_index("c")
    row_base = wid * ROWS_PER_W

    def chunk(t, carry):
        r0 = row_base + t * R_PER_CHUNK
        pltpu.sync_copy(x_hbm.at[pl.ds(r0, R_PER_CHUNK)], idx_v)
        handles = [
            pltpu.async_copy(table_hbm.at[idx_v.at[j]], rows_v.at[j], gsem)
            for j in range(R_PER_CHUNK)
        ]
        for h in handles:
            h.wait()
        pltpu.sync_copy(rows_v, out_hbm.at[pl.ds(r0, R_PER_CHUNK)])
        return carry

    lax.fori_loop(0, T_CHUNKS, chunk, 0)


@functools.partial(jax.jit)
def kernel(x, weight):
    x2d = x.reshape(B_TOTAL // IDX_MINOR, IDX_MINOR).astype(jnp.int32)
    mesh = plsc.VectorSubcoreMesh(core_axis_name="c", subcore_axis_name="s")
    run = pl.kernel(
        _gather_body,
        out_type=jax.ShapeDtypeStruct(
            (B_TOTAL // IDX_MINOR, IDX_MINOR, EMBED_DIM), jnp.float32),
        mesh=mesh,
        scratch_types=[
            pltpu.VMEM((R_PER_CHUNK, IDX_MINOR), jnp.int32),
            pltpu.VMEM((R_PER_CHUNK, IDX_MINOR, EMBED_DIM), jnp.float32),
            pltpu.SemaphoreType.DMA,
        ],
        compiler_params=pltpu.CompilerParams(use_tc_tiling_on_sc=False),
    )
    out = run(x2d, weight)
    return out.reshape(BATCH, SEQ_LEN, EMBED_DIM)


# speedup vs baseline: 1.0091x; 1.0091x over previous
import functools

import jax
import jax.numpy as jnp
from jax import lax
from jax.experimental import pallas as pl
from jax.experimental.pallas import tpu as pltpu
from jax.experimental.pallas import tpu_sc as plsc

VOCAB = 1000000
EMBED_DIM = 64
BATCH = 4096
SEQ_LEN = 200

NC = 2
NS = 16
NW = NC * NS
B_TOTAL = BATCH * SEQ_LEN
N_PER_W = B_TOTAL // NW
IDX_MINOR = 128
R_PER_CHUNK = 8
CHUNK = R_PER_CHUNK * IDX_MINOR
ROWS_PER_W = N_PER_W // IDX_MINOR
T_CHUNKS = ROWS_PER_W // R_PER_CHUNK


NBUF = 2
R_SUB = R_PER_CHUNK // NBUF


def _gather_body(x_hbm, table_hbm, out_hbm, idx_v, rows_v0, rows_v1,
                 gsem, wsem0, wsem1):
    wid = lax.axis_index("s") * NC + lax.axis_index("c")
    row_base = wid * ROWS_PER_W
    bufs = ((rows_v0, wsem0), (rows_v1, wsem1))

    def chunk(g, carry):
        r0 = row_base + g * R_PER_CHUNK
        pltpu.sync_copy(x_hbm.at[pl.ds(r0, R_PER_CHUNK)], idx_v)
        for b in range(NBUF):
            rows_v, wsem = bufs[b]
            dst = out_hbm.at[pl.ds(r0 + b * R_SUB, R_SUB)]

            @pl.when(g >= 1)
            def _wait_prev():
                pltpu.make_async_copy(rows_v, dst, wsem).wait()

            handles = [
                pltpu.async_copy(table_hbm.at[idx_v.at[b * R_SUB + k]],
                                 rows_v.at[k], gsem)
                for k in range(R_SUB)
            ]
            for h in handles:
                h.wait()
            pltpu.async_copy(rows_v, dst, wsem)
        return carry

    lax.fori_loop(0, T_CHUNKS, chunk, 0)
    for b in range(NBUF):
        rows_v, wsem = bufs[b]
        pltpu.make_async_copy(
            rows_v, out_hbm.at[pl.ds(row_base + b * R_SUB, R_SUB)], wsem
        ).wait()


@functools.partial(jax.jit)
def kernel(x, weight):
    x2d = x.reshape(B_TOTAL // IDX_MINOR, IDX_MINOR).astype(jnp.int32)
    mesh = plsc.VectorSubcoreMesh(core_axis_name="c", subcore_axis_name="s")
    run = pl.kernel(
        _gather_body,
        out_type=jax.ShapeDtypeStruct(
            (B_TOTAL // IDX_MINOR, IDX_MINOR, EMBED_DIM), jnp.float32),
        mesh=mesh,
        scratch_types=[
            pltpu.VMEM((R_PER_CHUNK, IDX_MINOR), jnp.int32),
            pltpu.VMEM((R_SUB, IDX_MINOR, EMBED_DIM), jnp.float32),
            pltpu.VMEM((R_SUB, IDX_MINOR, EMBED_DIM), jnp.float32),
            pltpu.SemaphoreType.DMA,
            pltpu.SemaphoreType.DMA,
            pltpu.SemaphoreType.DMA,
        ],
        compiler_params=pltpu.CompilerParams(use_tc_tiling_on_sc=False),
    )
    out = run(x2d, weight)
    return out.reshape(BATCH, SEQ_LEN, EMBED_DIM)


# speedup vs baseline: 1.0136x; 1.0045x over previous
import functools

import jax
import jax.numpy as jnp
from jax import lax
from jax.experimental import pallas as pl
from jax.experimental.pallas import tpu as pltpu
from jax.experimental.pallas import tpu_sc as plsc

VOCAB = 1000000
EMBED_DIM = 64
BATCH = 4096
SEQ_LEN = 200

NC = 2
NS = 16
NW = NC * NS
B_TOTAL = BATCH * SEQ_LEN
IDX_MINOR = 128
B_PER_W = BATCH // NW
ROWS_PER_BIG = 16
IDXROWS_PER_BIG = ROWS_PER_BIG * SEQ_LEN // IDX_MINOR
G_BIG = B_PER_W // ROWS_PER_BIG
SUBS = 5
STREAMS_PER_SUB = IDXROWS_PER_BIG // SUBS
SUB_LOOKUPS = STREAMS_PER_SUB * IDX_MINOR

SUB_PATTERNS = (
    ((0, 0, 200), (1, 0, 200), (2, 0, 200), (3, 0, 40)),
    ((3, 40, 160), (4, 0, 200), (5, 0, 200), (6, 0, 80)),
    ((6, 80, 120), (7, 0, 200), (8, 0, 200), (9, 0, 120)),
    ((9, 120, 80), (10, 0, 200), (11, 0, 200), (12, 0, 160)),
    ((12, 160, 40), (13, 0, 200), (14, 0, 200), (15, 0, 200)),
)


def _gather_body(x_hbm, table_hbm, out_hbm, idx_v, rows_v0, rows_v1,
                 gsem, wsem0, wsem1):
    wid = lax.axis_index("s") * NC + lax.axis_index("c")
    idxrow_base = wid * (G_BIG * IDXROWS_PER_BIG)
    batch_base = wid * B_PER_W
    bufs = ((rows_v0, wsem0), (rows_v1, wsem1))

    def _pieces(rows_v, out_base, s):
        off = 0
        for (br, s0, ns) in SUB_PATTERNS[s]:
            yield (rows_v.at[pl.ds(off, ns)],
                   out_hbm.at[out_base + br, pl.ds(s0, ns)])
            off += ns

    def big(g, carry):
        pltpu.sync_copy(
            x_hbm.at[pl.ds(idxrow_base + g * IDXROWS_PER_BIG,
                           IDXROWS_PER_BIG)],
            idx_v)
        out_base = batch_base + g * ROWS_PER_BIG
        for s in range(SUBS):
            rows_v, wsem = bufs[s % 2]

            def _wait_prev():
                for src, dst in _pieces(rows_v, out_base, s):
                    pltpu.make_async_copy(src, dst, wsem).wait()
            if s < 2:
                @pl.when(g >= 1)
                def _w():
                    _wait_prev()
            else:
                _wait_prev()

            handles = [
                pltpu.async_copy(
                    table_hbm.at[idx_v.at[SUBS * s + j]],
                    rows_v.at[pl.ds(IDX_MINOR * j, IDX_MINOR)], gsem)
                for j in range(STREAMS_PER_SUB)
            ]
            for h in handles:
                h.wait()
            for src, dst in _pieces(rows_v, out_base, s):
                pltpu.async_copy(src, dst, wsem)
        return carry

    lax.fori_loop(0, G_BIG, big, 0)
    last_base = batch_base + (G_BIG - 1) * ROWS_PER_BIG
    for s, (rows_v, wsem) in ((4, bufs[0]), (3, bufs[1])):
        for src, dst in _pieces(rows_v, last_base, s):
            pltpu.make_async_copy(src, dst, wsem).wait()


@functools.partial(jax.jit)
def kernel(x, weight):
    x2d = x.reshape(B_TOTAL // IDX_MINOR, IDX_MINOR).astype(jnp.int32)
    mesh = plsc.VectorSubcoreMesh(core_axis_name="c", subcore_axis_name="s")
    run = pl.kernel(
        _gather_body,
        out_type=jax.ShapeDtypeStruct((BATCH, SEQ_LEN, EMBED_DIM),
                                      jnp.float32),
        mesh=mesh,
        scratch_types=[
            pltpu.VMEM((IDXROWS_PER_BIG, IDX_MINOR), jnp.int32),
            pltpu.VMEM((SUB_LOOKUPS, EMBED_DIM), jnp.float32),
            pltpu.VMEM((SUB_LOOKUPS, EMBED_DIM), jnp.float32),
            pltpu.SemaphoreType.DMA,
            pltpu.SemaphoreType.DMA,
            pltpu.SemaphoreType.DMA,
        ],
        compiler_params=pltpu.CompilerParams(use_tc_tiling_on_sc=False),
    )
    return run(x2d, weight)
